# baseline (speedup 1.0000x reference)
import functools

import numpy as np
import jax
import jax.numpy as jnp
from jax import lax
from jax.experimental import pallas as pl
from jax.experimental.pallas import tpu as pltpu
from jax.experimental.pallas import tpu_sc as plsc


_RANGE_BASE = {"face": 0, "leftHand": 468, "pose": 489, "rightHand": 522,
               "root": 543}


def _gather_indices():
    hand_dfs = [0, 1, 2, 3, 4, 3, 2, 1, 0, 5, 6, 7, 8, 7, 6, 5, 0, 9, 10,
                11, 12, 11, 10, 9, 0, 13, 14, 15, 16, 15, 14, 13, 0, 17, 18,
                19, 20, 19, 18, 17, 0]
    order = ["root", "pose_11", "pose_13", "pose_15"]
    order += ["leftHand_%d" % i for i in hand_dfs]
    order += ["pose_15", "pose_13", "pose_11", "pose_12", "pose_14",
              "pose_16"]
    order += ["rightHand_%d" % i for i in hand_dfs]
    order += ["face_%d" % i for i in
              [61, 185, 40, 39, 37, 0, 267, 269, 270, 409, 291]]
    out = []
    for joint in order:
        kind = joint.split("_")[0]
        lid = 0 if kind == "root" else int(joint.split("_")[1])
        out.append(_RANGE_BASE[kind] + lid)
    return np.asarray(out, dtype=np.int32)


_IDX = _gather_indices()
_B, _T, _K, _C = 64, 200, 544, 3
_BT = _B * _T
_ROW_W = _K * _C
_OUT_W = len(_IDX) * _C
_L = 16
_TROWS = _ROW_W // _L

_SRC = (np.repeat(_IDX * _C, _C) + np.tile(np.arange(_C), len(_IDX)))
_NEED = np.unique(_SRC // _L)
_NROWS = len(_NEED)
_POS = {int(r): i for i, r in enumerate(_NEED)}

_NVOUT = 20
_STORE_OFF = [v * _L for v in range(_NVOUT - 1)] + [_OUT_W - _L]

_IOTA = np.arange(_L)


def _build_plan():
    plan = []
    pats = []
    for v in range(_NVOUT):
        words = _SRC[_STORE_OFF[v]:_STORE_OFF[v] + _L]
        rows = np.array([_POS[int(s) // _L] for s in words])
        lanes = np.array([int(s) % _L for s in words])
        contribs = []
        for p in sorted(set(rows.tolist())):
            sel = rows == p
            perm = np.where(sel, lanes, 0).astype(np.int32)
            mask = sel.astype(np.int32)
            is_id = bool(np.all(perm[sel] == _IOTA[sel]))
            pair = len(pats)
            pats.append(np.concatenate([perm, mask]))
            contribs.append((int(p), pair, is_id))
        plan.append(contribs)
    return plan, np.concatenate(pats).astype(np.int32)


_PLAN, _PATS = _build_plan()
_NPAIRS = len(_PATS) // (2 * _L)

_IDX_FULL = (np.arange(_BT, dtype=np.int32)[:, None] * _TROWS
             + _NEED[None, :].astype(np.int32)).reshape(-1)

_NWORKERS = 32
_ROWS_PER_W = _BT // _NWORKERS
_CHUNK = 40
_NCHUNKS = _ROWS_PER_W // _CHUNK


_GDNUMS = lax.GatherDimensionNumbers(
    offset_dims=(), collapsed_slice_dims=(0,), start_index_map=(0,))


def _shuffle(val, perm):
    return lax.gather(val, perm[:, None], _GDNUMS, slice_sizes=(1,),
                      mode=lax.GatherScatterMode.PROMISE_IN_BOUNDS)


def _body(table, idxs, pats, out, idx_v, gbuf, obuf, pats_v, sem):
    c = lax.axis_index("c")
    s = lax.axis_index("s")
    w = s * 2 + c
    pltpu.sync_copy(pats, pats_v)
    base_row = w * _ROWS_PER_W

    perms = [pats_v[pl.ds(i * 2 * _L, _L)] for i in range(_NPAIRS)]
    masks = [pats_v[pl.ds(i * 2 * _L + _L, _L)] > 0 for i in range(_NPAIRS)]

    def chunk(ci, carry):
        row0 = base_row + ci * _CHUNK
        pltpu.sync_copy(idxs.at[pl.ds(row0 * _NROWS, _CHUNK * _NROWS)],
                        idx_v)
        pltpu.async_copy(table.at[idx_v], gbuf, sem).wait()

        def rloop(r, inner):
            rb = r * _NROWS
            ob = r * _OUT_W
            rows = [gbuf[rb + p, :] for p in range(_NROWS)]
            for v in range(_NVOUT):
                acc = None
                for p, pair, is_id in _PLAN[v]:
                    shuf = rows[p] if is_id else _shuffle(rows[p],
                                                          perms[pair])
                    acc = shuf if acc is None else jnp.where(masks[pair],
                                                             shuf, acc)
                obuf[pl.ds(ob + _STORE_OFF[v], _L)] = acc
            return inner

        lax.fori_loop(0, _CHUNK, rloop, 0)
        pltpu.sync_copy(obuf,
                        out.at[pl.ds(row0 * _OUT_W, _CHUNK * _OUT_W)])
        return carry

    lax.fori_loop(0, _NCHUNKS, chunk, 0)


@jax.jit
def _run(table):
    mesh = plsc.VectorSubcoreMesh(core_axis_name="c", subcore_axis_name="s")
    fn = functools.partial(
        pl.kernel,
        mesh=mesh,
        compiler_params=pltpu.CompilerParams(use_tc_tiling_on_sc=False),
        out_type=jax.ShapeDtypeStruct((_BT * _OUT_W,), jnp.float32),
        scratch_types=[
            pltpu.VMEM((_CHUNK * _NROWS,), jnp.int32),
            pltpu.VMEM((_CHUNK * _NROWS, _L), jnp.float32),
            pltpu.VMEM((_CHUNK * _OUT_W,), jnp.float32),
            pltpu.VMEM((_NPAIRS * 2 * _L,), jnp.int32),
            pltpu.SemaphoreType.DMA,
        ],
    )(_body)
    return fn(table, jnp.asarray(_IDX_FULL), jnp.asarray(_PATS))


def kernel(keypoints):
    table = keypoints.reshape(_BT * _TROWS, _L)
    out = _run(table)
    return out.reshape(_B, _T, len(_IDX), _C)


# speedup vs baseline: 397.2250x; 397.2250x over previous
import functools

import numpy as np
import jax
import jax.numpy as jnp
from jax.experimental import pallas as pl
from jax.experimental.pallas import tpu as pltpu


_RANGE_BASE = {"face": 0, "leftHand": 468, "pose": 489, "rightHand": 522,
               "root": 543}


def _gather_indices():
    hand_dfs = [0, 1, 2, 3, 4, 3, 2, 1, 0, 5, 6, 7, 8, 7, 6, 5, 0, 9, 10,
                11, 12, 11, 10, 9, 0, 13, 14, 15, 16, 15, 14, 13, 0, 17, 18,
                19, 20, 19, 18, 17, 0]
    order = ["root", "pose_11", "pose_13", "pose_15"]
    order += ["leftHand_%d" % i for i in hand_dfs]
    order += ["pose_15", "pose_13", "pose_11", "pose_12", "pose_14",
              "pose_16"]
    order += ["rightHand_%d" % i for i in hand_dfs]
    order += ["face_%d" % i for i in
              [61, 185, 40, 39, 37, 0, 267, 269, 270, 409, 291]]
    out = []
    for joint in order:
        kind = joint.split("_")[0]
        lid = 0 if kind == "root" else int(joint.split("_")[1])
        out.append(_RANGE_BASE[kind] + lid)
    return np.asarray(out, dtype=np.int32)


_IDX = _gather_indices()
_B, _T, _K, _C = 64, 200, 544, 3
_NP = _B * _C
_NOUT = len(_IDX)

_SEL = np.zeros((_K, _NOUT), dtype=np.float32)
_SEL[_IDX, np.arange(_NOUT)] = 1.0

_PLANES_PER_STEP = 4


def _body(x_ref, s_ref, o_ref):
    s = s_ref[...]
    for p in range(_PLANES_PER_STEP):
        o_ref[p] = jax.lax.dot_general(
            x_ref[p], s, (((1,), (0,)), ((), ())),
            preferred_element_type=jnp.float32)


@jax.jit
def _run(xp):
    grid = (_NP // _PLANES_PER_STEP,)
    return pl.pallas_call(
        _body,
        grid=grid,
        in_specs=[
            pl.BlockSpec((_PLANES_PER_STEP, _T, _K), lambda g: (g, 0, 0)),
            pl.BlockSpec((_K, _NOUT), lambda g: (0, 0)),
        ],
        out_specs=pl.BlockSpec((_PLANES_PER_STEP, _T, _NOUT),
                               lambda g: (g, 0, 0)),
        out_shape=jax.ShapeDtypeStruct((_NP, _T, _NOUT), jnp.float32),
        compiler_params=pltpu.CompilerParams(
            dimension_semantics=("parallel",)),
    )(xp, jnp.asarray(_SEL))


def kernel(keypoints):
    xp = jnp.transpose(keypoints, (0, 3, 1, 2)).reshape(_NP, _T, _K)
    out = _run(xp)
    return jnp.transpose(out.reshape(_B, _C, _T, _NOUT), (0, 2, 3, 1))
